# baseline (speedup 1.0000x reference)
import functools

import jax
import jax.numpy as jnp
from jax import lax
from jax.experimental import pallas as pl
from jax.experimental.pallas import tpu as pltpu
from jax.experimental.pallas import tpu_sc as plsc

B = 4096
L = 200
V = 1000000
D = 64
H = 256
O = 64

NC = 2
NS = 16
NW = NC * NS
CHUNK = 104
LPAD = 2 * CHUNK
HR = B * 2
HR_PER_W = HR // NW
ROWS_PER_W = B // NW


def _pool_body(inp_hbm, emb_hbm, out_hbm, idx_v, buf_a, buf_b, stage,
               sem_a, sem_b):
    wid = lax.axis_index("s") * NC + lax.axis_index("c")
    hbase = wid * HR_PER_W
    rbase = wid * ROWS_PER_W

    pltpu.sync_copy(inp_hbm.at[pl.ds(hbase, HR_PER_W)], idx_v)

    pltpu.async_copy(emb_hbm.at[idx_v.at[0]], buf_a, sem_a)
    pltpu.async_copy(emb_hbm.at[idx_v.at[1]], buf_b, sem_b)

    def acc_chunk(buf, acc):
        def body(t, acc):
            return tuple(acc[j] + buf[t, pl.ds(16 * j, 16)] for j in range(4))
        return lax.fori_loop(0, CHUNK, body, acc, unroll=8)

    def row_loop(r, carry):
        zero = jnp.zeros((16,), jnp.float32)
        acc = (zero, zero, zero, zero)

        pltpu.make_async_copy(emb_hbm.at[idx_v.at[2 * r]], buf_a, sem_a).wait()
        acc = acc_chunk(buf_a, acc)

        @pl.when(r < ROWS_PER_W - 1)
        def _():
            pltpu.async_copy(emb_hbm.at[idx_v.at[2 * r + 2]], buf_a, sem_a)

        pltpu.make_async_copy(emb_hbm.at[idx_v.at[2 * r + 1]], buf_b,
                              sem_b).wait()
        acc = acc_chunk(buf_b, acc)

        @pl.when(r < ROWS_PER_W - 1)
        def _():
            pltpu.async_copy(emb_hbm.at[idx_v.at[2 * r + 3]], buf_b, sem_b)

        for j in range(4):
            stage[r, pl.ds(16 * j, 16)] = acc[j]
        return carry

    lax.fori_loop(0, ROWS_PER_W, row_loop, 0)

    pltpu.sync_copy(stage, out_hbm.at[pl.ds(rbase, ROWS_PER_W)])


_pool = functools.partial(
    pl.kernel,
    out_type=jax.ShapeDtypeStruct((B, D), jnp.float32),
    mesh=plsc.VectorSubcoreMesh(core_axis_name="c", subcore_axis_name="s"),
    scratch_types=[
        pltpu.VMEM((HR_PER_W, CHUNK), jnp.int32),
        pltpu.VMEM((CHUNK, D), jnp.float32),
        pltpu.VMEM((CHUNK, D), jnp.float32),
        pltpu.VMEM((ROWS_PER_W, D), jnp.float32),
        pltpu.SemaphoreType.DMA,
        pltpu.SemaphoreType.DMA,
    ],
    compiler_params=pltpu.CompilerParams(use_tc_tiling_on_sc=False),
)(_pool_body)


MLP_BLK = 512


def _mlp_body(sums_ref, inp_ref, sl_ref, pad_ref, emb0_ref, w1_ref, b1_ref,
              w2_ref, b2_ref, out_ref):
    is_pad = (inp_ref[...] == pad_ref[...]).astype(jnp.float32)
    cnt = jnp.sum(is_pad, axis=1, keepdims=True) + float(LPAD - L)
    avg = (sums_ref[...] - cnt * emb0_ref[...]) / sl_ref[...]
    h = jnp.dot(avg, w1_ref[...], preferred_element_type=jnp.float32)
    h = jnp.maximum(h + b1_ref[...], 0.0)
    out = jnp.dot(h, w2_ref[...], preferred_element_type=jnp.float32)
    out_ref[...] = out + b2_ref[...]


def _mlp(sums, inp, sl, pad, emb0, w1, b1, w2, b2):
    grid = (B // MLP_BLK,)
    return pl.pallas_call(
        _mlp_body,
        grid=grid,
        in_specs=[
            pl.BlockSpec((MLP_BLK, D), lambda i: (i, 0)),
            pl.BlockSpec((MLP_BLK, L), lambda i: (i, 0)),
            pl.BlockSpec((MLP_BLK, 1), lambda i: (i, 0)),
            pl.BlockSpec((1, 1), lambda i: (0, 0)),
            pl.BlockSpec((1, D), lambda i: (0, 0)),
            pl.BlockSpec((D, H), lambda i: (0, 0)),
            pl.BlockSpec((1, H), lambda i: (0, 0)),
            pl.BlockSpec((H, O), lambda i: (0, 0)),
            pl.BlockSpec((1, O), lambda i: (0, 0)),
        ],
        out_specs=pl.BlockSpec((MLP_BLK, O), lambda i: (i, 0)),
        out_shape=jax.ShapeDtypeStruct((B, O), jnp.float32),
    )(sums, inp, sl, pad, emb0, w1, b1, w2, b2)


def kernel(input, seq_lengths, pad_id, emb, W1, b1, W2, b2):
    pad_arr = jnp.asarray(pad_id, jnp.int32)
    inp_half = input.reshape(HR, L // 2)
    inp_pad = jnp.pad(inp_half, ((0, 0), (0, CHUNK - L // 2)),
                      constant_values=pad_arr)
    sums = _pool(inp_pad, emb)
    sl = seq_lengths.astype(jnp.float32).reshape(B, 1)
    emb0 = lax.dynamic_slice_in_dim(emb, pad_arr, 1, axis=0)
    return _mlp(sums, input, sl, pad_arr.reshape(1, 1), emb0,
                W1, b1.reshape(1, H), W2, b2.reshape(1, O))
